# baseline (speedup 1.0000x reference)
import functools

import jax
import jax.numpy as jnp
from jax import lax
from jax.experimental import pallas as pl
from jax.experimental.pallas import tpu as pltpu
from jax.experimental.pallas import tpu_sc as plsc

VOCAB_N = 100000
NCORES = 2
NSUB = 16
LANES = 16
HALF = VOCAB_N // NCORES
SLICE = 3136
HPAD = SLICE * NSUB
N_PRED = 16384 * 50
N_CAPT = 16384 * 200
PRED_PER_TILE = N_PRED // NSUB
CAPT_PER_TILE = N_CAPT // NSUB
CHUNK = 6400

_mesh = plsc.VectorSubcoreMesh(core_axis_name="c", subcore_axis_name="s")


@functools.partial(
    pl.kernel,
    out_type=jax.ShapeDtypeStruct((NCORES * NSUB, 2, LANES), jnp.int32),
    mesh=_mesh,
    scratch_types=[
        pltpu.VMEM((HPAD,), jnp.int32),
        pltpu.VMEM((CHUNK,), jnp.int32),
        pltpu.VMEM((SLICE,), jnp.int32),
        pltpu.VMEM((SLICE,), jnp.int32),
        pltpu.VMEM((2, LANES), jnp.int32),
        pltpu.VMEM_SHARED((NSUB, HPAD), jnp.int32),
    ],
    compiler_params=pltpu.CompilerParams(use_tc_tiling_on_sc=False,
                                         needs_layout_passes=False),
)
def _vocab_usage_sc(preds_hbm, capts_hbm, out_hbm,
                    pres, tbuf, mbuf, macc, cbuf, shared):
    core = lax.axis_index("c")
    sub = lax.axis_index("s")
    wid = core * NSUB + sub
    base = core * HALF
    zeros16 = jnp.zeros((LANES,), jnp.int32)
    ones16 = jnp.ones((LANES,), jnp.int32)

    def _zero_pres():
        def _z(i, carry):
            pres[pl.ds(i * LANES, LANES)] = zeros16
            return carry
        lax.fori_loop(0, HPAD // LANES, _z, 0)

    def _scatter(src_hbm, per_tile):
        tile_base = sub * per_tile

        def _chunk(ch, carry):
            pltpu.sync_copy(src_hbm.at[pl.ds(tile_base + ch * CHUNK, CHUNK)],
                            tbuf)

            def _vec(i, c2):
                tok = tbuf[pl.ds(i * LANES, LANES)]
                loc = tok - base
                msk = (loc >= 0) & (loc < HALF)
                locc = jnp.minimum(jnp.maximum(loc, 0), HPAD - 1)
                plsc.store_scatter(pres, [locc], ones16, mask=msk)
                return c2
            return lax.fori_loop(0, CHUNK // LANES, _vec, carry)
        lax.fori_loop(0, per_tile // CHUNK, _chunk, 0)

    sl_start = sub * SLICE

    def _merge_count(inp):
        pltpu.sync_copy(shared.at[0, pl.ds(sl_start, SLICE)], macc)

        def _merge_tile(t, carry):
            pltpu.sync_copy(shared.at[t, pl.ds(sl_start, SLICE)], mbuf)

            def _orv(j, c2):
                macc[pl.ds(j * LANES, LANES)] = (
                    macc[pl.ds(j * LANES, LANES)]
                    | mbuf[pl.ds(j * LANES, LANES)])
                return c2
            return lax.fori_loop(0, SLICE // LANES, _orv, carry)
        lax.fori_loop(1, NSUB, _merge_tile, 0)

        def _cnt(j, cv):
            return cv + (macc[pl.ds(j * LANES, LANES)] != 0).astype(jnp.int32)
        cbuf[inp] = lax.fori_loop(0, SLICE // LANES, _cnt, zeros16)

    _zero_pres()
    _scatter(preds_hbm, PRED_PER_TILE)
    pltpu.sync_copy(pres, shared.at[sub])
    plsc.subcore_barrier()
    _merge_count(0)
    _zero_pres()
    _scatter(capts_hbm, CAPT_PER_TILE)
    plsc.subcore_barrier()
    pltpu.sync_copy(pres, shared.at[sub])
    plsc.subcore_barrier()
    _merge_count(1)

    pltpu.sync_copy(cbuf, out_hbm.at[wid])


def kernel(preds, captions):
    parts = _vocab_usage_sc(preds.reshape(-1), captions.reshape(-1))
    n_pred = parts[:, 0, :].sum().astype(jnp.float32)
    n_capt = parts[:, 1, :].sum().astype(jnp.float32)
    return jnp.where(n_capt > 0, n_pred / jnp.maximum(n_capt, 1.0),
                     jnp.float32(0.0))


# speedup vs baseline: 2.7447x; 2.7447x over previous
import functools

import jax
import jax.numpy as jnp
from jax import lax
from jax.experimental import pallas as pl
from jax.experimental.pallas import tpu as pltpu
from jax.experimental.pallas import tpu_sc as plsc

VOCAB_N = 100000
NCORES = 2
NSUB = 16
LANES = 16
HALF = VOCAB_N // NCORES
SLICE = 3136
HPAD = SLICE * NSUB
N_PRED = 16384 * 50
N_CAPT = 16384 * 200
PRED_PER_TILE = N_PRED // NSUB
CAPT_PER_TILE = N_CAPT // NSUB
CHUNK = 6400

_mesh = plsc.VectorSubcoreMesh(core_axis_name="c", subcore_axis_name="s")


@functools.partial(
    pl.kernel,
    out_type=jax.ShapeDtypeStruct((NCORES * NSUB, 2, LANES), jnp.int32),
    mesh=_mesh,
    scratch_types=[
        pltpu.VMEM((HPAD,), jnp.int32),
        pltpu.VMEM((2, CHUNK), jnp.int32),
        pltpu.VMEM((2, LANES), jnp.int32),
        pltpu.VMEM_SHARED((NSUB, HPAD), jnp.int32),
        pltpu.SemaphoreType.DMA,
        pltpu.SemaphoreType.DMA,
        pltpu.SemaphoreType.DMA,
    ],
    compiler_params=pltpu.CompilerParams(use_tc_tiling_on_sc=False,
                                         needs_layout_passes=False),
)
def _vocab_usage_sc(preds_hbm, capts_hbm, out_hbm,
                    pres, tbuf, cbuf, shared, sem_a, sem_b, sem_m):
    core = lax.axis_index("c")
    sub = lax.axis_index("s")
    wid = core * NSUB + sub
    base = core * HALF
    zeros16 = jnp.zeros((LANES,), jnp.int32)
    ones16 = jnp.ones((LANES,), jnp.int32)
    sl_start = sub * SLICE

    def _zero_pres():
        @plsc.parallel_loop(0, HPAD // LANES, unroll=8)
        def _z(i):
            pres[pl.ds(i * LANES, LANES)] = zeros16

    def _scatter(src_hbm, per_tile):
        tile_base = sub * per_tile
        nch = per_tile // CHUNK
        sems = (sem_a, sem_b)
        desc = [None, None]
        desc[0] = pltpu.async_copy(src_hbm.at[pl.ds(tile_base, CHUNK)],
                                   tbuf.at[0], sems[0])
        for ch in range(nch):
            b = ch % 2
            if ch + 1 < nch:
                nb = (ch + 1) % 2
                desc[nb] = pltpu.async_copy(
                    src_hbm.at[pl.ds(tile_base + (ch + 1) * CHUNK, CHUNK)],
                    tbuf.at[nb], sems[nb])
            desc[b].wait()

            @plsc.parallel_loop(0, CHUNK // LANES, unroll=8)
            def _v(i):
                tok = tbuf[b, pl.ds(i * LANES, LANES)]
                loc = tok - base
                msk = loc.astype(jnp.uint32) < jnp.uint32(HALF)
                plsc.store_scatter(pres, [loc], ones16, mask=msk)

    def _merge_count(inp):
        descs = [pltpu.async_copy(shared.at[t, pl.ds(sl_start, SLICE)],
                                  pres.at[pl.ds(t * SLICE, SLICE)], sem_m)
                 for t in range(NSUB)]
        for d in descs:
            d.wait()

        @plsc.parallel_loop(0, SLICE // LANES, unroll=4, carry=zeros16)
        def _cnt(j, cv):
            acc = pres[pl.ds(j * LANES, LANES)]
            for t in range(1, NSUB):
                acc = acc | pres[pl.ds(t * SLICE + j * LANES, LANES)]
            return cv + (acc != 0).astype(jnp.int32)
        cbuf[inp] = _cnt

    _zero_pres()
    _scatter(preds_hbm, PRED_PER_TILE)
    pltpu.sync_copy(pres, shared.at[sub])
    plsc.subcore_barrier()
    _merge_count(0)
    plsc.subcore_barrier()
    _zero_pres()
    _scatter(capts_hbm, CAPT_PER_TILE)
    pltpu.sync_copy(pres, shared.at[sub])
    plsc.subcore_barrier()
    _merge_count(1)

    pltpu.sync_copy(cbuf, out_hbm.at[wid])


def kernel(preds, captions):
    parts = _vocab_usage_sc(preds.reshape(-1), captions.reshape(-1))
    n_pred = parts[:, 0, :].sum().astype(jnp.float32)
    n_capt = parts[:, 1, :].sum().astype(jnp.float32)
    return jnp.where(n_capt > 0, n_pred / jnp.maximum(n_capt, 1.0),
                     jnp.float32(0.0))
